# baseline (speedup 1.0000x reference)
import functools

import jax
import jax.numpy as jnp
from jax import lax
from jax.experimental import pallas as pl
from jax.experimental.pallas import tpu as pltpu
from jax.experimental.pallas import tpu_sc as plsc

N_NODES = 100000
D_FEAT = 128
NUM_TYPES = 64

NC = 2
NS = 16
NW = NC * NS

C = 200
NCHUNKS = N_NODES // C
IDXW = 100
NGATHER = C // IDXW

_mesh = plsc.VectorSubcoreMesh(core_axis_name="c", subcore_axis_name="s")


@functools.partial(
    pl.kernel,
    out_type=jax.ShapeDtypeStruct((N_NODES, D_FEAT), jnp.float32),
    mesh=_mesh,
    compiler_params=pltpu.CompilerParams(needs_layout_passes=False),
    scratch_types=[
        pltpu.VMEM_SHARED((NUM_TYPES, D_FEAT), jnp.float32),
        pltpu.VMEM((2, C, D_FEAT), jnp.float32),
        pltpu.VMEM((2, C, D_FEAT), jnp.float32),
        pltpu.VMEM((2, NGATHER, IDXW), jnp.int32),
        pltpu.SemaphoreType.DMA((2,)),
        pltpu.SemaphoreType.DMA((2,)),
        pltpu.SemaphoreType.DMA((2,)),
    ],
)
def _sc_embed_add(x_hbm, types_hbm, table_hbm, out_hbm,
                  table_v, xbuf, rowbuf, tbuf, sin_x, sin_t, sout):
    wid = lax.axis_index("s") * NC + lax.axis_index("c")

    @pl.when(lax.axis_index("s") == 0)
    def _stage_table():
        pltpu.sync_copy(table_hbm, table_v)

    plsc.subcore_barrier()

    n_my = (NCHUNKS - wid + NW - 1) // NW

    def in_copies(c, b):
        return (
            pltpu.make_async_copy(
                x_hbm.at[pl.ds(c * C, C), :], xbuf.at[b], sin_x.at[b]),
            pltpu.make_async_copy(types_hbm.at[c], tbuf.at[b], sin_t.at[b]),
        )

    def out_copy(c, b):
        return pltpu.make_async_copy(
            xbuf.at[b], out_hbm.at[pl.ds(c * C, C), :], sout.at[b])

    for cp in in_copies(wid, 0):
        cp.start()

    def chunk_body(i, carry):
        c = wid + i * NW
        b = lax.rem(i, 2)
        for cp in in_copies(c, b):
            cp.wait()

        for j in range(NGATHER):
            pltpu.sync_copy(
                table_v.at[tbuf.at[b, j]],
                rowbuf.at[b, pl.ds(j * IDXW, IDXW), :],
            )

        nb = 1 - b

        @pl.when(i + 1 < n_my)
        def _prefetch():
            @pl.when(i >= 1)
            def _drain_prev_out():
                out_copy(c - NW, nb).wait()

            for cp in in_copies(c + NW, nb):
                cp.start()

        def add_body(r, carry2):
            for j in range(0, D_FEAT, 16):
                vals = rowbuf[b, r, pl.ds(j, 16)]
                plsc.addupdate(xbuf.at[b, r, pl.ds(j, 16)], vals)
            return carry2

        lax.fori_loop(0, C, add_body, 0)

        out_copy(c, b).start()
        return carry

    lax.fori_loop(0, n_my, chunk_body, 0)

    @pl.when(n_my >= 2)
    def _drain_m2():
        i = n_my - 2
        out_copy(wid + i * NW, lax.rem(i, 2)).wait()

    i_last = n_my - 1
    out_copy(wid + i_last * NW, lax.rem(i_last, 2)).wait()


def kernel(x, node_types, type_table):
    types_r = node_types.astype(jnp.int32).reshape(NCHUNKS, NGATHER, IDXW)
    return _sc_embed_add(x, types_r, type_table)


# speedup vs baseline: 1.5436x; 1.5436x over previous
import functools

import jax
import jax.numpy as jnp
import numpy as np
from jax import lax
from jax.experimental import pallas as pl
from jax.experimental.pallas import tpu as pltpu
from jax.experimental.pallas import tpu_sc as plsc

N_NODES = 100000
D_FEAT = 128
NUM_TYPES = 64

NC = 2
NS = 16
NW = NC * NS

C = 240
NCHUNKS = -(-N_NODES // C)
IDXW = 120
NGATHER = C // IDXW
MAXCH = -(-NCHUNKS // NW)
NPAIRS = -(-MAXCH // 2)

_mesh = plsc.VectorSubcoreMesh(core_axis_name="c", subcore_axis_name="s")


@functools.partial(
    pl.kernel,
    out_type=jax.ShapeDtypeStruct((N_NODES, D_FEAT), jnp.float32),
    mesh=_mesh,
    compiler_params=pltpu.CompilerParams(needs_layout_passes=False),
    scratch_types=[
        pltpu.VMEM_SHARED((NUM_TYPES, D_FEAT), jnp.float32),
        pltpu.VMEM((C, D_FEAT), jnp.float32),
        pltpu.VMEM((C, D_FEAT), jnp.float32),
        pltpu.VMEM((C, D_FEAT), jnp.float32),
        pltpu.VMEM((C, D_FEAT), jnp.float32),
        pltpu.VMEM((NGATHER, IDXW), jnp.int32),
        pltpu.VMEM((NGATHER, IDXW), jnp.int32),
        pltpu.SemaphoreType.DMA((2,)),
        pltpu.SemaphoreType.DMA((2,)),
        pltpu.SemaphoreType.DMA((2,)),
        pltpu.SemaphoreType.DMA((2,)),
    ],
)
def _sc_embed_add(x_hbm, types_hbm, table_hbm, out_hbm,
                  table_v, xb0, xb1, rb0, rb1, tb0, tb1,
                  sin_x, sin_t, sg, sout):
    xb = (xb0, xb1)
    rb = (rb0, rb1)
    tb = (tb0, tb1)
    wid = lax.axis_index("s") * NC + lax.axis_index("c")

    @pl.when(lax.axis_index("s") == 0)
    def _stage_table():
        pltpu.sync_copy(table_hbm, table_v)

    plsc.subcore_barrier()

    n_my = (NCHUNKS - wid + NW - 1) // NW

    def xbase(i):
        return jnp.minimum((wid + i * NW) * C, N_NODES - C)

    def in_x(i, b):
        return pltpu.make_async_copy(
            x_hbm.at[pl.ds(xbase(i), C), :], xb[b], sin_x.at[b])

    def in_t(i, b):
        return pltpu.make_async_copy(
            types_hbm.at[wid + i * NW], tb[b], sin_t.at[b])

    def gathers(b):
        return [
            pltpu.make_async_copy(
                table_v.at[tb[b].at[j]],
                rb[b].at[pl.ds(j * IDXW, IDXW), :],
                sg.at[b],
            )
            for j in range(NGATHER)
        ]

    def out_copy(i, b):
        return pltpu.make_async_copy(
            xb[b], out_hbm.at[pl.ds(xbase(i), C), :], sout.at[b])

    in_x(0, 0).start()
    in_t(0, 0).start()
    in_t(0, 0).wait()
    for g in gathers(0):
        g.start()

    def half(i, b):
        nb = 1 - b

        @pl.when(i + 1 < n_my)
        def _prefetch():
            @pl.when(i >= 1)
            def _drain_prev_out():
                out_copy(i - 1, nb).wait()

            in_x(i + 1, nb).start()
            in_t(i + 1, nb).start()
            in_t(i + 1, nb).wait()
            for g in gathers(nb):
                g.start()

        in_x(i, b).wait()
        for g in gathers(b):
            g.wait()

        def add_body(r, carry2):
            for j in range(0, D_FEAT, 16):
                vals = rb[b][r, pl.ds(j, 16)]
                plsc.addupdate(xb[b].at[r, pl.ds(j, 16)], vals)
            return carry2

        lax.fori_loop(0, C, add_body, 0)
        out_copy(i, b).start()

    def pair_body(p, carry):
        i0 = p * 2

        @pl.when(i0 < n_my)
        def _a():
            half(i0, 0)

        @pl.when(i0 + 1 < n_my)
        def _b():
            half(i0 + 1, 1)

        return carry

    lax.fori_loop(0, NPAIRS, pair_body, 0)

    for b in (0, 1):
        @pl.when(n_my >= b + 1)
        def _drain(b=b):
            i = jnp.where(lax.rem(n_my - 1, 2) == b, n_my - 1, n_my - 2)
            out_copy(i, b).wait()


def kernel(x, node_types, type_table):
    node_types = node_types.astype(jnp.int32)
    bases = np.minimum(np.arange(NCHUNKS) * C, N_NODES - C)
    idx = bases[:, None] + np.arange(C)
    types_chunks = node_types[idx].reshape(NCHUNKS, NGATHER, IDXW)
    return _sc_embed_add(x, types_chunks, type_table)


# speedup vs baseline: 2.4624x; 1.5953x over previous
import functools

import jax
import jax.numpy as jnp
from jax import lax
from jax.experimental import pallas as pl
from jax.experimental.pallas import tpu as pltpu
from jax.experimental.pallas import tpu_sc as plsc

N_NODES = 100000
D_FEAT = 128
NUM_TYPES = 64

NC = 2
NS = 16
NW = NC * NS

C = 400
NCHUNKS = N_NODES // C
IDXW = 100
NGATHER = C // IDXW
MAXCH = -(-NCHUNKS // NW)
NPAIRS = -(-MAXCH // 2)

_mesh = plsc.VectorSubcoreMesh(core_axis_name="c", subcore_axis_name="s")


@functools.partial(
    pl.kernel,
    out_type=jax.ShapeDtypeStruct((N_NODES, D_FEAT), jnp.float32),
    mesh=_mesh,
    compiler_params=pltpu.CompilerParams(needs_layout_passes=False),
    scratch_types=[
        pltpu.VMEM_SHARED((NUM_TYPES, D_FEAT), jnp.float32),
        pltpu.VMEM((C, D_FEAT), jnp.float32),
        pltpu.VMEM((C, D_FEAT), jnp.float32),
        pltpu.VMEM((NGATHER, IDXW), jnp.int32),
        pltpu.VMEM((NGATHER, IDXW), jnp.int32),
        pltpu.SemaphoreType.DMA((2,)),
        pltpu.SemaphoreType.DMA((2,)),
        pltpu.SemaphoreType.DMA((2,)),
        pltpu.SemaphoreType.DMA((2,)),
    ],
)
def _sc_embed_add(x_hbm, types_hbm, table_hbm, out_hbm,
                  table_v, xb0, xb1, tb0, tb1,
                  sin_x, sin_t, sg, sout):
    xb = (xb0, xb1)
    tb = (tb0, tb1)
    wid = lax.axis_index("s") * NC + lax.axis_index("c")

    @pl.when(lax.axis_index("s") == 0)
    def _stage_table():
        pltpu.sync_copy(table_hbm, table_v)

    plsc.subcore_barrier()

    n_my = (NCHUNKS - wid + NW - 1) // NW

    def in_x(i, b):
        c = wid + i * NW
        return pltpu.make_async_copy(
            x_hbm.at[pl.ds(c * C, C), :], xb[b], sin_x.at[b])

    def in_t(i, b):
        return pltpu.make_async_copy(
            types_hbm.at[wid + i * NW], tb[b], sin_t.at[b])

    def gather_adds(b):
        return [
            pltpu.async_copy(
                table_v.at[tb[b].at[j]],
                xb[b].at[pl.ds(j * IDXW, IDXW), :],
                sg.at[b],
                add=True,
            )
            for j in range(NGATHER)
        ]

    def wait_gathers(b):
        for j in range(NGATHER):
            pltpu.make_async_copy(
                table_v.at[tb[b].at[j]],
                xb[b].at[pl.ds(j * IDXW, IDXW), :],
                sg.at[b],
            ).wait()

    def out_copy(i, b):
        c = wid + i * NW
        return pltpu.make_async_copy(
            xb[b], out_hbm.at[pl.ds(c * C, C), :], sout.at[b])

    in_x(0, 0).start()
    in_t(0, 0).start()

    def half(i, b):
        nb = 1 - b

        @pl.when(i + 1 < n_my)
        def _prefetch():
            @pl.when(i >= 1)
            def _drain_prev_out():
                out_copy(i - 1, nb).wait()

            in_x(i + 1, nb).start()
            in_t(i + 1, nb).start()

        in_x(i, b).wait()
        in_t(i, b).wait()
        gather_adds(b)
        wait_gathers(b)
        out_copy(i, b).start()

    def pair_body(p, carry):
        i0 = p * 2

        @pl.when(i0 < n_my)
        def _a():
            half(i0, 0)

        @pl.when(i0 + 1 < n_my)
        def _b():
            half(i0 + 1, 1)

        return carry

    lax.fori_loop(0, NPAIRS, pair_body, 0)

    for b in (0, 1):
        @pl.when(n_my >= b + 1)
        def _drain(b=b):
            i = jnp.where(lax.rem(n_my - 1, 2) == b, n_my - 1, n_my - 2)
            out_copy(i, b).wait()


def kernel(x, node_types, type_table):
    types_r = node_types.astype(jnp.int32).reshape(NCHUNKS, NGATHER, IDXW)
    return _sc_embed_add(x, types_r, type_table)


# speedup vs baseline: 2.5053x; 1.0174x over previous
import functools

import jax
import jax.numpy as jnp
from jax import lax
from jax.experimental import pallas as pl
from jax.experimental.pallas import tpu as pltpu
from jax.experimental.pallas import tpu_sc as plsc

N_NODES = 100000
D_FEAT = 128
NUM_TYPES = 64

NC = 2
NS = 16
NW = NC * NS

C = 400
NCHUNKS = N_NODES // C
IDXW = 100
NGATHER = C // IDXW
MAXCH = -(-NCHUNKS // NW)
NPAIRS = -(-MAXCH // 2)

_mesh = plsc.VectorSubcoreMesh(core_axis_name="c", subcore_axis_name="s")


@functools.partial(
    pl.kernel,
    out_type=jax.ShapeDtypeStruct((N_NODES, D_FEAT), jnp.float32),
    mesh=_mesh,
    compiler_params=pltpu.CompilerParams(needs_layout_passes=False),
    scratch_types=[
        pltpu.VMEM_SHARED((NUM_TYPES, D_FEAT), jnp.float32),
        pltpu.VMEM((C, D_FEAT), jnp.float32),
        pltpu.VMEM((C, D_FEAT), jnp.float32),
        pltpu.VMEM((NGATHER, IDXW), jnp.int32),
        pltpu.VMEM((NGATHER, IDXW), jnp.int32),
        pltpu.SemaphoreType.DMA((2,)),
        pltpu.SemaphoreType.DMA((2,)),
        pltpu.SemaphoreType.DMA((2,)),
        pltpu.SemaphoreType.DMA((2,)),
    ],
)
def _sc_embed_add(x_hbm, types_hbm, table_hbm, out_hbm,
                  table_v, xb0, xb1, tb0, tb1,
                  sin_x, sin_t, sg, sout):
    xb = (xb0, xb1)
    tb = (tb0, tb1)
    wid = lax.axis_index("s") * NC + lax.axis_index("c")

    n_my = (NCHUNKS - wid + NW - 1) // NW

    def in_x(i, b):
        c = wid + i * NW
        return pltpu.make_async_copy(
            x_hbm.at[pl.ds(c * C, C), :], xb[b], sin_x.at[b])

    def in_t(i, b):
        return pltpu.make_async_copy(
            types_hbm.at[wid + i * NW], tb[b], sin_t.at[b])

    def gather_adds(b):
        return [
            pltpu.async_copy(
                table_v.at[tb[b].at[j]],
                xb[b].at[pl.ds(j * IDXW, IDXW), :],
                sg.at[b],
                add=True,
            )
            for j in range(NGATHER)
        ]

    def wait_gathers(b):
        for j in range(NGATHER):
            pltpu.make_async_copy(
                table_v.at[tb[b].at[j]],
                xb[b].at[pl.ds(j * IDXW, IDXW), :],
                sg.at[b],
            ).wait()

    def out_copy(i, b):
        c = wid + i * NW
        return pltpu.make_async_copy(
            xb[b], out_hbm.at[pl.ds(c * C, C), :], sout.at[b])

    in_x(0, 0).start()
    in_t(0, 0).start()

    @pl.when(lax.axis_index("s") == 0)
    def _stage_table():
        pltpu.sync_copy(table_hbm, table_v)

    plsc.subcore_barrier()

    def half(i, b):
        nb = 1 - b

        @pl.when(i + 1 < n_my)
        def _prefetch():
            @pl.when(i >= 1)
            def _drain_prev_out():
                out_copy(i - 1, nb).wait()

            in_x(i + 1, nb).start()
            in_t(i + 1, nb).start()

        in_x(i, b).wait()
        in_t(i, b).wait()
        gather_adds(b)
        wait_gathers(b)
        out_copy(i, b).start()

    def pair_body(p, carry):
        i0 = p * 2

        @pl.when(i0 < n_my)
        def _a():
            half(i0, 0)

        @pl.when(i0 + 1 < n_my)
        def _b():
            half(i0 + 1, 1)

        return carry

    lax.fori_loop(0, NPAIRS, pair_body, 0)

    for b in (0, 1):
        @pl.when(n_my >= b + 1)
        def _drain(b=b):
            i = jnp.where(lax.rem(n_my - 1, 2) == b, n_my - 1, n_my - 2)
            out_copy(i, b).wait()


def kernel(x, node_types, type_table):
    types_r = node_types.astype(jnp.int32).reshape(NCHUNKS, NGATHER, IDXW)
    return _sc_embed_add(x, types_r, type_table)
